# baseline (speedup 1.0000x reference)
import functools

import jax
import jax.numpy as jnp
from jax import lax
from jax.experimental import pallas as pl
from jax.experimental.pallas import tpu as pltpu
from jax.experimental.pallas import tpu_sc as plsc


_LANE = 128
_SUB = 8
_BLOCK_N = 2048


def _argmin_body(zt_ref, cn2t_ref, c2_ref, idx_ref):
    zt = zt_ref[...]
    cn2t = cn2t_ref[...]
    bn = zt.shape[1]
    k = cn2t.shape[1]
    mmt = lax.dot_general(
        cn2t, zt, (((0,), (0,)), ((), ())),
        preferred_element_type=jnp.float32,
    )
    z2l = jnp.sum(zt * zt, axis=0, keepdims=True)
    z2b = jnp.broadcast_to(z2l, (_SUB, bn))
    run_min = None
    run_idx = None
    sub_iota = lax.broadcasted_iota(jnp.int32, (_SUB, bn), 0)
    for c in range(k // _SUB):
        s = (z2b + c2_ref[c * _SUB:(c + 1) * _SUB, :]) \
            + mmt[c * _SUB:(c + 1) * _SUB, :]
        cur_idx = sub_iota + (c * _SUB)
        if run_min is None:
            run_min, run_idx = s, cur_idx
        else:
            better = s < run_min
            run_min = jnp.where(better, s, run_min)
            run_idx = jnp.where(better, cur_idx, run_idx)
    m = jnp.min(run_min, axis=0, keepdims=True)
    idxv = jnp.min(jnp.where(run_min == m, run_idx, jnp.int32(k)),
                   axis=0, keepdims=True)
    for t in range(bn // _LANE):
        idx_ref[t] = jnp.broadcast_to(
            idxv[:, t * _LANE:(t + 1) * _LANE], (_SUB, _LANE))


def _compute_indices(zt, cn2t, c2col, n_half, off_blocks):
    d, n = zt.shape
    k = cn2t.shape[1]
    grid = n_half // _BLOCK_N
    tiles_per_blk = _BLOCK_N // _LANE
    return pl.pallas_call(
        _argmin_body,
        grid=(grid,),
        in_specs=[
            pl.BlockSpec((d, _BLOCK_N), lambda i: (0, i + off_blocks)),
            pl.BlockSpec((d, k), lambda i: (0, 0)),
            pl.BlockSpec((k, 1), lambda i: (0, 0)),
        ],
        out_specs=pl.BlockSpec((tiles_per_blk, _SUB, _LANE),
                               lambda i: (i, 0, 0)),
        out_shape=jax.ShapeDtypeStruct((n_half // _LANE, _SUB, _LANE),
                                       jnp.int32),
    )(zt, cn2t, c2col)


_NC = 2
_NS = 16
_NW = _NC * _NS
_CH = 128


def _make_sc_gather(k, n, w):
    b_per_w = n // _NW
    n_ch = b_per_w // _CH
    mesh = plsc.VectorSubcoreMesh(core_axis_name="c", subcore_axis_name="s")

    @functools.partial(
        pl.kernel,
        out_type=jax.ShapeDtypeStruct((n, _LANE), jnp.float32),
        mesh=mesh,
        scratch_types=[
            pltpu.VMEM((n_ch, 1, _CH), jnp.int32),
            pltpu.VMEM((2, _CH, w), jnp.float32),
            pltpu.SemaphoreType.DMA,
            pltpu.SemaphoreType.DMA,
        ],
        compiler_params=pltpu.CompilerParams(use_tc_tiling_on_sc=False),
    )
    def gather_kernel(table_hbm, idx_hbm, out_hbm, idx_v, rows_v, sem0, sem1):
        wid = lax.axis_index("s") * _NC + lax.axis_index("c")
        base = wid * b_per_w
        pltpu.sync_copy(
            idx_hbm.at[pl.ds(n_ch * wid, n_ch), pl.ds(0, 1)], idx_v)
        sems = (sem0, sem1)
        cps = [None, None]
        cps[0] = pltpu.async_copy(
            table_hbm.at[idx_v.at[0, 0]], rows_v.at[0], sems[0])
        for j in range(n_ch):
            s = j % 2
            if j + 1 < n_ch:
                cps[(j + 1) % 2] = pltpu.async_copy(
                    table_hbm.at[idx_v.at[j + 1, 0]], rows_v.at[(j + 1) % 2],
                    sems[(j + 1) % 2])
            cps[s].wait()
            pltpu.sync_copy(
                rows_v.at[s],
                out_hbm.at[pl.ds(base + j * _CH, _CH), pl.ds(0, w)])

    return gather_kernel


_FIN_BN = 2048


def _finish_body(qp0_ref, qp1_ref, out_ref):
    w = out_ref.shape[0] // 2
    gh = pl.num_programs(0) // 2
    first = pl.program_id(0) < gh
    x = jnp.where(first, qp0_ref[:, 0:w], qp1_ref[:, 0:w])
    xu = lax.bitcast_convert_type(x, jnp.uint32)
    xt = xu.T
    lo = lax.bitcast_convert_type(
        lax.shift_left(xt, jnp.uint32(16)), jnp.float32)
    hi = lax.bitcast_convert_type(
        xt & jnp.uint32(0xFFFF0000), jnp.float32)
    out_ref[0:w, :] = lo
    out_ref[w:2 * w, :] = hi


def _unpack_transposed(qp0, qp1, d):
    nh = qp0.shape[0]
    n = 2 * nh
    gh = nh // _FIN_BN
    return pl.pallas_call(
        _finish_body,
        grid=(2 * gh,),
        in_specs=[
            pl.BlockSpec((_FIN_BN, _LANE),
                         lambda i: (jnp.minimum(i, gh - 1), 0)),
            pl.BlockSpec((_FIN_BN, _LANE),
                         lambda i: (jnp.maximum(i - gh, 0), 0)),
        ],
        out_specs=pl.BlockSpec((d, _FIN_BN), lambda i: (0, i)),
        out_shape=jax.ShapeDtypeStruct((d, n), jnp.float32),
    )(qp0, qp1)


def kernel(latents, codebook):
    n, d = latents.shape
    k = codebook.shape[0]
    c2 = jnp.sum(codebook ** 2, axis=1)
    c2col = c2.reshape(k, 1)
    zt = latents.T
    cn2t = (-2.0 * codebook).T
    cb_bf = codebook.astype(jnp.bfloat16)
    pairs = cb_bf.reshape(k, 2, d // 2).transpose(0, 2, 1)
    table = lax.bitcast_convert_type(pairs, jnp.float32)
    nh = n // 2
    sc = _make_sc_gather(k, nh, d // 2)
    idx0 = _compute_indices(zt, cn2t, c2col, nh, 0)
    qp0 = sc(table, idx0)
    idx1 = _compute_indices(zt, cn2t, c2col, nh, nh // _BLOCK_N)
    qp1 = sc(table, idx1)
    return _unpack_transposed(qp0, qp1, d).T


# speedup vs baseline: 1.0104x; 1.0104x over previous
import functools

import jax
import jax.numpy as jnp
from jax import lax
from jax.experimental import pallas as pl
from jax.experimental.pallas import tpu as pltpu
from jax.experimental.pallas import tpu_sc as plsc


_LANE = 128
_SUB = 8
_BLOCK_N = 4096


def _argmin_body(zt_ref, cn2t_ref, c2_ref, idx_ref):
    zt = zt_ref[...]
    cn2t = cn2t_ref[...]
    bn = zt.shape[1]
    k = cn2t.shape[1]
    mmt = lax.dot_general(
        cn2t, zt, (((0,), (0,)), ((), ())),
        preferred_element_type=jnp.float32,
    )
    z2l = jnp.sum(zt * zt, axis=0, keepdims=True)
    z2b = jnp.broadcast_to(z2l, (_SUB, bn))
    run_min = None
    run_idx = None
    sub_iota = lax.broadcasted_iota(jnp.int32, (_SUB, bn), 0)
    for c in range(k // _SUB):
        s = (z2b + c2_ref[c * _SUB:(c + 1) * _SUB, :]) \
            + mmt[c * _SUB:(c + 1) * _SUB, :]
        cur_idx = sub_iota + (c * _SUB)
        if run_min is None:
            run_min, run_idx = s, cur_idx
        else:
            better = s < run_min
            run_min = jnp.where(better, s, run_min)
            run_idx = jnp.where(better, cur_idx, run_idx)
    m = jnp.min(run_min, axis=0, keepdims=True)
    idxv = jnp.min(jnp.where(run_min == m, run_idx, jnp.int32(k)),
                   axis=0, keepdims=True)
    for t in range(bn // _LANE):
        idx_ref[t] = jnp.broadcast_to(
            idxv[:, t * _LANE:(t + 1) * _LANE], (_SUB, _LANE))


def _compute_indices(zt, cn2t, c2col, n_half, off_blocks):
    d, n = zt.shape
    k = cn2t.shape[1]
    grid = n_half // _BLOCK_N
    tiles_per_blk = _BLOCK_N // _LANE
    return pl.pallas_call(
        _argmin_body,
        grid=(grid,),
        in_specs=[
            pl.BlockSpec((d, _BLOCK_N), lambda i: (0, i + off_blocks)),
            pl.BlockSpec((d, k), lambda i: (0, 0)),
            pl.BlockSpec((k, 1), lambda i: (0, 0)),
        ],
        out_specs=pl.BlockSpec((tiles_per_blk, _SUB, _LANE),
                               lambda i: (i, 0, 0)),
        out_shape=jax.ShapeDtypeStruct((n_half // _LANE, _SUB, _LANE),
                                       jnp.int32),
    )(zt, cn2t, c2col)


_NC = 2
_NS = 16
_NW = _NC * _NS
_CH = 128


def _make_sc_gather(k, n, w):
    b_per_w = n // _NW
    n_ch = b_per_w // _CH
    mesh = plsc.VectorSubcoreMesh(core_axis_name="c", subcore_axis_name="s")

    @functools.partial(
        pl.kernel,
        out_type=jax.ShapeDtypeStruct((n, _LANE), jnp.float32),
        mesh=mesh,
        scratch_types=[
            pltpu.VMEM((n_ch, 1, _CH), jnp.int32),
            pltpu.VMEM((2, _CH, w), jnp.float32),
            pltpu.SemaphoreType.DMA,
            pltpu.SemaphoreType.DMA,
        ],
        compiler_params=pltpu.CompilerParams(
            use_tc_tiling_on_sc=False, skip_device_barrier=True),
    )
    def gather_kernel(table_hbm, idx_hbm, out_hbm, idx_v, rows_v, sem0, sem1):
        wid = lax.axis_index("s") * _NC + lax.axis_index("c")
        base = wid * b_per_w
        pltpu.sync_copy(
            idx_hbm.at[pl.ds(n_ch * wid, n_ch), pl.ds(0, 1)], idx_v)
        sems = (sem0, sem1)
        cps = [None, None]
        cps[0] = pltpu.async_copy(
            table_hbm.at[idx_v.at[0, 0]], rows_v.at[0], sems[0])
        for j in range(n_ch):
            s = j % 2
            if j + 1 < n_ch:
                cps[(j + 1) % 2] = pltpu.async_copy(
                    table_hbm.at[idx_v.at[j + 1, 0]], rows_v.at[(j + 1) % 2],
                    sems[(j + 1) % 2])
            cps[s].wait()
            pltpu.sync_copy(
                rows_v.at[s],
                out_hbm.at[pl.ds(base + j * _CH, _CH), pl.ds(0, w)])

    return gather_kernel


_FIN_BN = 2048


def _finish_body(qp0_ref, qp1_ref, out_ref):
    w = out_ref.shape[0] // 2
    gh = pl.num_programs(0) // 2
    first = pl.program_id(0) < gh
    x = jnp.where(first, qp0_ref[:, 0:w], qp1_ref[:, 0:w])
    xu = lax.bitcast_convert_type(x, jnp.uint32)
    xt = xu.T
    lo = lax.bitcast_convert_type(
        lax.shift_left(xt, jnp.uint32(16)), jnp.float32)
    hi = lax.bitcast_convert_type(
        xt & jnp.uint32(0xFFFF0000), jnp.float32)
    out_ref[0:w, :] = lo
    out_ref[w:2 * w, :] = hi


def _unpack_transposed(qp0, qp1, d):
    nh = qp0.shape[0]
    n = 2 * nh
    gh = nh // _FIN_BN
    return pl.pallas_call(
        _finish_body,
        grid=(2 * gh,),
        in_specs=[
            pl.BlockSpec((_FIN_BN, _LANE),
                         lambda i: (jnp.minimum(i, gh - 1), 0)),
            pl.BlockSpec((_FIN_BN, _LANE),
                         lambda i: (jnp.maximum(i - gh, 0), 0)),
        ],
        out_specs=pl.BlockSpec((d, _FIN_BN), lambda i: (0, i)),
        out_shape=jax.ShapeDtypeStruct((d, n), jnp.float32),
    )(qp0, qp1)


def kernel(latents, codebook):
    n, d = latents.shape
    k = codebook.shape[0]
    c2 = jnp.sum(codebook ** 2, axis=1)
    c2col = c2.reshape(k, 1)
    zt = latents.T
    cn2t = (-2.0 * codebook).T
    cb_bf = codebook.astype(jnp.bfloat16)
    pairs = cb_bf.reshape(k, 2, d // 2).transpose(0, 2, 1)
    table = lax.bitcast_convert_type(pairs, jnp.float32)
    nh = n // 2
    sc = _make_sc_gather(k, nh, d // 2)
    idx0 = _compute_indices(zt, cn2t, c2col, nh, 0)
    qp0 = sc(table, idx0)
    idx1 = _compute_indices(zt, cn2t, c2col, nh, nh // _BLOCK_N)
    qp1 = sc(table, idx1)
    return _unpack_transposed(qp0, qp1, d).T


# speedup vs baseline: 1.0565x; 1.0457x over previous
import functools

import jax
import jax.numpy as jnp
from jax import lax
from jax.experimental import pallas as pl
from jax.experimental.pallas import tpu as pltpu
from jax.experimental.pallas import tpu_sc as plsc


_LANE = 128
_SUB = 8
_BLOCK_N = 4096


def _argmin_body(zt_ref, cn2t_ref, c2_ref, idx_ref):
    zt = zt_ref[...]
    cn2t = cn2t_ref[...]
    bn = zt.shape[1]
    k = cn2t.shape[1]
    mmt = lax.dot_general(
        cn2t, zt, (((0,), (0,)), ((), ())),
        preferred_element_type=jnp.float32,
    )
    z2l = jnp.sum(zt * zt, axis=0, keepdims=True)
    z2b = jnp.broadcast_to(z2l, (_SUB, bn))
    run_min = None
    run_idx = None
    sub_iota = lax.broadcasted_iota(jnp.int32, (_SUB, bn), 0)
    for c in range(k // _SUB):
        s = (z2b + c2_ref[c * _SUB:(c + 1) * _SUB, :]) \
            + mmt[c * _SUB:(c + 1) * _SUB, :]
        cur_idx = sub_iota + (c * _SUB)
        if run_min is None:
            run_min, run_idx = s, cur_idx
        else:
            better = s < run_min
            run_min = jnp.where(better, s, run_min)
            run_idx = jnp.where(better, cur_idx, run_idx)
    m = jnp.min(run_min, axis=0, keepdims=True)
    idxv = jnp.min(jnp.where(run_min == m, run_idx, jnp.int32(k)),
                   axis=0, keepdims=True)
    for t in range(bn // _LANE):
        idx_ref[t] = jnp.broadcast_to(
            idxv[:, t * _LANE:(t + 1) * _LANE], (_SUB, _LANE))


def _compute_indices(zt, cn2t, c2col, n_half, off_blocks):
    d, n = zt.shape
    k = cn2t.shape[1]
    grid = n_half // _BLOCK_N
    tiles_per_blk = _BLOCK_N // _LANE
    return pl.pallas_call(
        _argmin_body,
        grid=(grid,),
        in_specs=[
            pl.BlockSpec((d, _BLOCK_N), lambda i: (0, i + off_blocks)),
            pl.BlockSpec((d, k), lambda i: (0, 0)),
            pl.BlockSpec((k, 1), lambda i: (0, 0)),
        ],
        out_specs=pl.BlockSpec((tiles_per_blk, _SUB, _LANE),
                               lambda i: (i, 0, 0)),
        out_shape=jax.ShapeDtypeStruct((n_half // _LANE, _SUB, _LANE),
                                       jnp.int32),
    )(zt, cn2t, c2col)


_NC = 2
_NS = 16
_NW = _NC * _NS
_CH = 128


def _make_sc_gather(k, n, w):
    b_per_w = n // _NW
    n_ch = b_per_w // _CH
    mesh = plsc.VectorSubcoreMesh(core_axis_name="c", subcore_axis_name="s")

    @functools.partial(
        pl.kernel,
        out_type=jax.ShapeDtypeStruct((n, _LANE), jnp.float32),
        mesh=mesh,
        scratch_types=[
            pltpu.VMEM((n_ch, 1, _CH), jnp.int32),
            pltpu.VMEM((2, _CH, w), jnp.float32),
            pltpu.SemaphoreType.DMA,
            pltpu.SemaphoreType.DMA,
        ],
        compiler_params=pltpu.CompilerParams(
            use_tc_tiling_on_sc=False, skip_device_barrier=True),
    )
    def gather_kernel(table_hbm, idx_hbm, out_hbm, idx_v, rows_v, sem0, sem1):
        wid = lax.axis_index("s") * _NC + lax.axis_index("c")
        base = wid * b_per_w
        pltpu.sync_copy(
            idx_hbm.at[pl.ds(n_ch * wid, n_ch), pl.ds(0, 1)], idx_v)
        sems = (sem0, sem1)
        cps = [None, None]
        cps[0] = pltpu.async_copy(
            table_hbm.at[idx_v.at[0, 0]], rows_v.at[0], sems[0])
        for j in range(n_ch):
            s = j % 2
            if j + 1 < n_ch:
                cps[(j + 1) % 2] = pltpu.async_copy(
                    table_hbm.at[idx_v.at[j + 1, 0]], rows_v.at[(j + 1) % 2],
                    sems[(j + 1) % 2])
            cps[s].wait()
            pltpu.sync_copy(
                rows_v.at[s],
                out_hbm.at[pl.ds(base + j * _CH, _CH), pl.ds(0, w)])

    return gather_kernel


_FIN_BN = 2048


def _finish_body(qp_ref, prev_ref, out_ref):
    del prev_ref
    w = out_ref.shape[0] // 2
    x = qp_ref[:, 0:w]
    xu = lax.bitcast_convert_type(x, jnp.uint32)
    xt = xu.T
    lo = lax.bitcast_convert_type(
        lax.shift_left(xt, jnp.uint32(16)), jnp.float32)
    hi = lax.bitcast_convert_type(
        xt & jnp.uint32(0xFFFF0000), jnp.float32)
    out_ref[0:w, :] = lo
    out_ref[w:2 * w, :] = hi


def _unpack_half(qp_h, prev, d, n, off_blocks):
    gh = qp_h.shape[0] // _FIN_BN
    if prev is None:
        def body0(qp_ref, out_ref):
            _finish_body(qp_ref, None, out_ref)
        return pl.pallas_call(
            body0,
            grid=(gh,),
            in_specs=[pl.BlockSpec((_FIN_BN, _LANE), lambda i: (i, 0))],
            out_specs=pl.BlockSpec((d, _FIN_BN),
                                   lambda i: (0, i + off_blocks)),
            out_shape=jax.ShapeDtypeStruct((d, n), jnp.float32),
        )(qp_h)
    return pl.pallas_call(
        _finish_body,
        grid=(gh,),
        in_specs=[
            pl.BlockSpec((_FIN_BN, _LANE), lambda i: (i, 0)),
            pl.BlockSpec(memory_space=pl.ANY),
        ],
        out_specs=pl.BlockSpec((d, _FIN_BN), lambda i: (0, i + off_blocks)),
        out_shape=jax.ShapeDtypeStruct((d, n), jnp.float32),
        input_output_aliases={1: 0},
    )(qp_h, prev)


def kernel(latents, codebook):
    n, d = latents.shape
    k = codebook.shape[0]
    c2 = jnp.sum(codebook ** 2, axis=1)
    c2col = c2.reshape(k, 1)
    zt = latents.T
    cn2t = (-2.0 * codebook).T
    cb_bf = codebook.astype(jnp.bfloat16)
    pairs = cb_bf.reshape(k, 2, d // 2).transpose(0, 2, 1)
    table = lax.bitcast_convert_type(pairs, jnp.float32)
    nh = n // 2
    sc = _make_sc_gather(k, nh, d // 2)
    idx0 = _compute_indices(zt, cn2t, c2col, nh, 0)
    qp0 = sc(table, idx0)
    idx1 = _compute_indices(zt, cn2t, c2col, nh, nh // _BLOCK_N)
    qp1 = sc(table, idx1)
    gh = nh // _FIN_BN
    half0 = _unpack_half(qp0, None, d, n, 0)
    full = _unpack_half(qp1, half0, d, n, gh)
    return full.T
